# baseline (speedup 1.0000x reference)
import functools
import math

import jax
import jax.numpy as jnp
import numpy as np
from jax import lax
from jax.experimental import pallas as pl
from jax.experimental.pallas import tpu as pltpu
from jax.experimental.pallas import tpu_sc as plsc

B = 16384
L = 10
D = 64
NC = 2
NS = 16
NW = NC * NS
PPW = B // NW
CH = 32
NCHUNK = PPW // CH
NR = L - 1
SUB = 4
VOCAB_ROWS = 1000000
REL_PAD_SUBROW = 256 * SUB


def _pos_enc() -> np.ndarray:
    pe = np.zeros((L, D), dtype=np.float32)
    position = np.arange(0, L, dtype=np.float32)[:, None]
    div_term = np.exp(np.arange(0, D, 2).astype(np.float32) * (-math.log(10000.0) / D))
    pe[:, 0::2] = np.sin(position * div_term)
    pe[:, 1::2] = np.cos(position * div_term)
    return pe


_NSUB = CH * L * SUB
_RSUB = CH * NR * SUB
_NSTR = _NSUB // 128
_RSTR_FULL = _RSUB // 128
_RSTR_REM = (_RSUB % 128) // 64


def _sc_body(paths_hbm, rels_hbm, pshift_hbm, node_hbm, relpad_hbm, out_hbm,
             pv, rv, sv, nrows, rrows, obuf, sem):
    c = lax.axis_index("c")
    s = lax.axis_index("s")
    wid = s * NC + c

    def chunk(k, carry):
        pbase = pl.multiple_of(wid * PPW + k * CH, CH)
        noff = pl.multiple_of(pbase * (L * SUB), 8)
        roff = pl.multiple_of(pbase * (NR * SUB), 8)
        pltpu.sync_copy(paths_hbm.at[pl.ds(noff, _NSUB)], pv)
        pltpu.sync_copy(rels_hbm.at[pl.ds(roff, _RSUB)], rv)
        pltpu.sync_copy(pshift_hbm.at[pl.ds(roff, _RSUB)], sv)

        def redirect(i, carry2):
            sl = pl.ds(i * 16, 16)
            rv[sl] = jnp.where(sv[sl] != 0, rv[sl], REL_PAD_SUBROW)
            return carry2

        lax.fori_loop(0, _RSUB // 16, redirect, 0)

        cps = []
        for j in range(_NSTR):
            cps.append(pltpu.async_copy(
                node_hbm.at[pv.at[pl.ds(j * 128, 128)]],
                nrows.at[pl.ds(j * 128, 128)], sem))
        for j in range(_RSTR_FULL):
            cps.append(pltpu.async_copy(
                relpad_hbm.at[rv.at[pl.ds(j * 128, 128)]],
                rrows.at[pl.ds(j * 128, 128)], sem))
        for j in range(_RSTR_REM):
            off = _RSTR_FULL * 128 + j * 64
            cps.append(pltpu.async_copy(
                relpad_hbm.at[rv.at[pl.ds(off, 64)]],
                rrows.at[pl.ds(off, 64)], sem))
        for cp in cps:
            cp.wait()

        def acc_path(p, carry2):
            bn = p * (L * SUB)
            br = p * (NR * SUB)
            for q in range(SUB):
                a = nrows[bn + q, :]
                for l in range(1, L):
                    a = a + nrows[bn + l * SUB + q, :]
                for j in range(NR):
                    a = a + rrows[br + j * SUB + q, :]
                obuf[p, pl.ds(q * 16, 16)] = a
            return carry2

        lax.fori_loop(0, CH, acc_path, 0)
        pltpu.sync_copy(obuf, out_hbm.at[pl.ds(pbase, CH)])
        return carry

    lax.fori_loop(0, NCHUNK, chunk, 0)


@jax.jit
def _sc_sums(paths2d, rels2d, pshift, node_table, relpad):
    mesh = plsc.VectorSubcoreMesh(core_axis_name="c", subcore_axis_name="s")
    f = pl.kernel(
        _sc_body,
        out_type=jax.ShapeDtypeStruct((B, D), jnp.float32),
        mesh=mesh,
        scratch_types=[
            pltpu.VMEM((_NSUB,), jnp.int32),
            pltpu.VMEM((_RSUB,), jnp.int32),
            pltpu.VMEM((_RSUB,), jnp.int32),
            pltpu.VMEM((_NSUB, 16), jnp.float32),
            pltpu.VMEM((_RSUB, 16), jnp.float32),
            pltpu.VMEM((CH, D), jnp.float32),
            pltpu.SemaphoreType.DMA,
        ],
        compiler_params=pltpu.CompilerParams(use_tc_tiling_on_sc=False),
    )
    return f(paths2d, rels2d, pshift, node_table, relpad)


def _tc_body(sums_ref, paths_ref, row0_ref, pe_ref, w1_ref, b1_ref,
             w2_ref, b2_ref, out_ref):
    maskf = (paths_ref[...] != 0).astype(jnp.float32)
    dsum = jnp.sum(maskf, axis=1, keepdims=True)
    denom = jnp.maximum(dsum, 1.0)
    cnt0 = jnp.float32(L) - dsum
    pe_pool = jnp.dot(maskf, pe_ref[...], preferred_element_type=jnp.float32)
    pooled = (sums_ref[...] + pe_pool - cnt0 * row0_ref[...]) / denom
    h = jnp.maximum(
        jnp.dot(pooled, w1_ref[...], preferred_element_type=jnp.float32)
        + b1_ref[...], 0.0)
    out_ref[...] = (
        jnp.dot(h, w2_ref[...], preferred_element_type=jnp.float32)
        + b2_ref[...])


@jax.jit
def _tc_mlp(sums, paths_pad, row0, pe_pad, W1, b1, W2, b2):
    blk = 512
    grid = B // blk
    return pl.pallas_call(
        _tc_body,
        grid=(grid,),
        in_specs=[
            pl.BlockSpec((blk, D), lambda i: (i, 0)),
            pl.BlockSpec((blk, 16), lambda i: (i, 0)),
            pl.BlockSpec((1, D), lambda i: (0, 0)),
            pl.BlockSpec((16, D), lambda i: (0, 0)),
            pl.BlockSpec((D, D), lambda i: (0, 0)),
            pl.BlockSpec((1, D), lambda i: (0, 0)),
            pl.BlockSpec((D, D), lambda i: (0, 0)),
            pl.BlockSpec((1, D), lambda i: (0, 0)),
        ],
        out_specs=pl.BlockSpec((blk, D), lambda i: (i, 0)),
        out_shape=jax.ShapeDtypeStruct((B, D), jnp.float32),
    )(sums, paths_pad, row0, pe_pad, W1, b1, W2, b2)


def kernel(paths, rels, node_table, rel_table, W1, b1, W2, b2):
    paths = paths.astype(jnp.int32)
    rels = rels.astype(jnp.int32)
    sub = jnp.arange(SUB, dtype=jnp.int32)
    paths4 = (paths.reshape(B * L, 1) * SUB + sub).reshape(B * L * SUB)
    rels4 = (rels.reshape(B * NR, 1) * SUB + sub).reshape(B * NR * SUB)
    pshift4 = jnp.broadcast_to(
        paths[:, 1:].reshape(B * NR, 1), (B * NR, SUB)).reshape(B * NR * SUB)
    node16 = node_table.reshape(VOCAB_ROWS * SUB, 16)
    relpad = jnp.concatenate(
        [rel_table, jnp.zeros((8, D), dtype=rel_table.dtype)], axis=0)
    relpad16 = relpad.reshape(264 * SUB, 16)
    sums = _sc_sums(paths4, rels4, pshift4, node16, relpad16)

    paths_pad = jnp.concatenate(
        [paths, jnp.zeros((B, 16 - L), dtype=jnp.int32)], axis=1)
    pe_pad = jnp.asarray(np.pad(_pos_enc(), ((0, 16 - L), (0, 0))))
    return _tc_mlp(sums, paths_pad, node_table[0:1], pe_pad,
                   W1, b1.reshape(1, D), W2, b2.reshape(1, D))


# speedup vs baseline: 1.4060x; 1.4060x over previous
import functools
import math

import jax
import jax.numpy as jnp
import numpy as np
from jax import lax
from jax.experimental import pallas as pl
from jax.experimental.pallas import tpu as pltpu
from jax.experimental.pallas import tpu_sc as plsc

B = 16384
L = 10
D = 64
DP = 128
NC = 2
NS = 16
NW = NC * NS
PPW = B // NW
CH = 32
NCHUNK = PPW // CH
NR = L - 1
REL_PAD_ROW = 256


def _pos_enc() -> np.ndarray:
    pe = np.zeros((L, D), dtype=np.float32)
    position = np.arange(0, L, dtype=np.float32)[:, None]
    div_term = np.exp(np.arange(0, D, 2).astype(np.float32) * (-math.log(10000.0) / D))
    pe[:, 0::2] = np.sin(position * div_term)
    pe[:, 1::2] = np.cos(position * div_term)
    return pe


_NIDX = CH * L
_RIDX = CH * NR


def _idx_slices(total):
    out = []
    off = 0
    while off < total:
        n = min(128, total - off)
        out.append((off, n))
        off += n
    return out


def _sc_body(paths_hbm, rels_hbm, pshift_hbm, node_hbm, relpad_hbm, out_hbm,
             pv, rv, sv, nrows, rrows, obuf, sem):
    c = lax.axis_index("c")
    s = lax.axis_index("s")
    wid = s * NC + c

    def chunk(k, carry):
        pbase = pl.multiple_of(wid * PPW + k * CH, CH)
        noff = pl.multiple_of(pbase * L, 8)
        roff = pl.multiple_of(pbase * NR, 8)
        pltpu.sync_copy(paths_hbm.at[pl.ds(noff, _NIDX)], pv)
        pltpu.sync_copy(rels_hbm.at[pl.ds(roff, _RIDX)], rv)
        pltpu.sync_copy(pshift_hbm.at[pl.ds(roff, _RIDX)], sv)

        def redirect(i, carry2):
            sl = pl.ds(i * 16, 16)
            rv[sl] = jnp.where(sv[sl] != 0, rv[sl], REL_PAD_ROW)
            return carry2

        lax.fori_loop(0, _RIDX // 16, redirect, 0)

        cps = []
        for off, n in _idx_slices(_NIDX):
            cps.append(pltpu.async_copy(
                node_hbm.at[pv.at[pl.ds(off, n)]],
                nrows.at[pl.ds(off, n)], sem))
        for off, n in _idx_slices(_RIDX):
            cps.append(pltpu.async_copy(
                relpad_hbm.at[rv.at[pl.ds(off, n)]],
                rrows.at[pl.ds(off, n)], sem))
        for cp in cps:
            cp.wait()

        def acc_path(p, carry2):
            bn = p * L
            br = p * NR
            for q in range(D // 16):
                sl = pl.ds(q * 16, 16)
                a = nrows[bn, sl]
                for l in range(1, L):
                    a = a + nrows[bn + l, sl]
                for j in range(NR):
                    a = a + rrows[br + j, sl]
                obuf[p, sl] = a
            return carry2

        lax.fori_loop(0, CH, acc_path, 0)
        pltpu.sync_copy(obuf, out_hbm.at[pl.ds(pbase, CH)])
        return carry

    lax.fori_loop(0, NCHUNK, chunk, 0)


@jax.jit
def _sc_sums(paths_f, rels_f, pshift_f, node128, relpad128):
    mesh = plsc.VectorSubcoreMesh(core_axis_name="c", subcore_axis_name="s")
    f = pl.kernel(
        _sc_body,
        out_type=jax.ShapeDtypeStruct((B, DP), jnp.float32),
        mesh=mesh,
        scratch_types=[
            pltpu.VMEM((_NIDX,), jnp.int32),
            pltpu.VMEM((_RIDX,), jnp.int32),
            pltpu.VMEM((_RIDX,), jnp.int32),
            pltpu.VMEM((_NIDX, DP), jnp.float32),
            pltpu.VMEM((_RIDX, DP), jnp.float32),
            pltpu.VMEM((CH, DP), jnp.float32),
            pltpu.SemaphoreType.DMA,
        ],
        compiler_params=pltpu.CompilerParams(use_tc_tiling_on_sc=True),
    )
    return f(paths_f, rels_f, pshift_f, node128, relpad128)


def _tc_body(sums_ref, paths_ref, row0_ref, pe_ref, w1_ref, b1_ref,
             w2_ref, b2_ref, out_ref):
    maskf = (paths_ref[...] != 0).astype(jnp.float32)
    dsum = jnp.sum(maskf, axis=1, keepdims=True)
    denom = jnp.maximum(dsum, 1.0)
    cnt0 = jnp.float32(L) - dsum
    pe_pool = jnp.dot(maskf, pe_ref[...], preferred_element_type=jnp.float32)
    pooled = (sums_ref[...][:, :D] + pe_pool - cnt0 * row0_ref[...]) / denom
    h = jnp.maximum(
        jnp.dot(pooled, w1_ref[...], preferred_element_type=jnp.float32)
        + b1_ref[...], 0.0)
    out_ref[...] = (
        jnp.dot(h, w2_ref[...], preferred_element_type=jnp.float32)
        + b2_ref[...])


@jax.jit
def _tc_mlp(sums, paths_pad, row0, pe_pad, W1, b1, W2, b2):
    blk = 512
    grid = B // blk
    return pl.pallas_call(
        _tc_body,
        grid=(grid,),
        in_specs=[
            pl.BlockSpec((blk, DP), lambda i: (i, 0)),
            pl.BlockSpec((blk, 16), lambda i: (i, 0)),
            pl.BlockSpec((1, D), lambda i: (0, 0)),
            pl.BlockSpec((16, D), lambda i: (0, 0)),
            pl.BlockSpec((D, D), lambda i: (0, 0)),
            pl.BlockSpec((1, D), lambda i: (0, 0)),
            pl.BlockSpec((D, D), lambda i: (0, 0)),
            pl.BlockSpec((1, D), lambda i: (0, 0)),
        ],
        out_specs=pl.BlockSpec((blk, D), lambda i: (i, 0)),
        out_shape=jax.ShapeDtypeStruct((B, D), jnp.float32),
    )(sums, paths_pad, row0, pe_pad, W1, b1, W2, b2)


def kernel(paths, rels, node_table, rel_table, W1, b1, W2, b2):
    paths = paths.astype(jnp.int32)
    rels = rels.astype(jnp.int32)
    paths_f = paths.reshape(B * L)
    rels_f = rels.reshape(B * NR)
    pshift_f = paths[:, 1:].reshape(B * NR)
    node128 = jnp.pad(node_table, ((0, 0), (0, DP - D)))
    relpad128 = jnp.pad(rel_table, ((0, 8), (0, DP - D)))
    sums = _sc_sums(paths_f, rels_f, pshift_f, node128, relpad128)

    paths_pad = jnp.concatenate(
        [paths, jnp.zeros((B, 16 - L), dtype=jnp.int32)], axis=1)
    pe_pad = jnp.asarray(np.pad(_pos_enc(), ((0, 16 - L), (0, 0))))
    return _tc_mlp(sums, paths_pad, node_table[0:1], pe_pad,
                   W1, b1.reshape(1, D), W2, b2.reshape(1, D))
